# baseline (speedup 1.0000x reference)
import functools
import math

import jax
import jax.numpy as jnp
from jax import lax
from jax.experimental import pallas as pl
from jax.experimental.pallas import tpu as pltpu
from jax.experimental.pallas import tpu_sc as plsc

CUTOFF = 10.0
LOG2 = math.log(2.0)


LOG2E = 1.4426950408889634

_EXP2C = (-0.0009475536455669522, 0.009210875560850372,
          -0.055299607117477535, 0.24017948948888648,
          -0.6931432016425585, 0.9999999452374342)
_LOG2C = (-0.00887469665141809, 0.05027750737105334,
          -0.13453425419918055, 0.2392316629725045,
          -0.34599601243315514, 0.47832354486763984,
          -0.7211146144033584, 1.4426867778259647,
          4.886358020325664e-08)


_COSC = (1.7295049974733663e-09, -2.7094490891269423e-07,
         2.4771702330757988e-05, -0.0013887904388839413,
         0.04166651546784112, -0.49999991826057477,
         0.9999999954856433)


def _poly(coeffs, x):
    acc = jnp.full_like(x, coeffs[0])
    for c in coeffs[1:]:
        acc = acc * x + c
    return acc


def _ssp(v):
    u = jnp.minimum(jnp.abs(v) * LOG2E, 30.0)
    n = u.astype(jnp.int32)
    f = u - n.astype(jnp.float32)
    scale = lax.bitcast_convert_type(
        lax.shift_left(127 - n, 23), jnp.float32)
    t = _poly(_EXP2C, f) * scale
    g = _poly(_LOG2C, t) * LOG2
    return jnp.maximum(v, 0.0) + g - LOG2


def _dotT(a, b):
    return lax.dot_general(a, b, (((1,), (1,)), ((), ())),
                           preferred_element_type=jnp.float32)


def _xh_body(x_ref, w_ref, out_ref):
    out_ref[...] = _dotT(x_ref[...], w_ref[...])


def _compute_xh(x, lin1_w):
    N, H = x.shape
    F = lin1_w.shape[0]
    return pl.pallas_call(
        _xh_body,
        out_shape=jax.ShapeDtypeStruct((N, F), jnp.float32),
    )(x, lin1_w)


def _filter_body(ea_ref, el_ref, w1_ref, b1_ref, w2_ref, b2_ref, out_ref):
    ea = ea_ref[...]
    el = el_ref[...]
    h = _ssp(_dotT(ea, w1_ref[...]) + b1_ref[...])
    w = _dotT(h, w2_ref[...]) + b2_ref[...]
    z = el * (math.pi / CUTOFF)
    c = 0.5 * (_poly(_COSC, z * z) + 1.0)
    c = c * (el <= CUTOFF).astype(jnp.float32) * (el >= 0.0).astype(jnp.float32)
    out_ref[...] = w * c


def _compute_filter(edge_attr, el2d, w1, b1, w2, b2, eoff, ecnt):
    E, G = edge_attr.shape
    F = w2.shape[0]
    EB = 2560 if ecnt % 2560 == 0 else ecnt
    grid = (ecnt // EB,)
    boff = eoff // EB
    assert eoff % EB == 0
    return pl.pallas_call(
        _filter_body,
        grid=grid,
        in_specs=[
            pl.BlockSpec((EB, G), lambda i: (i + boff, 0)),
            pl.BlockSpec((EB, 1), lambda i: (i + boff, 0)),
            pl.BlockSpec((F, G), lambda i: (0, 0)),
            pl.BlockSpec((1, F), lambda i: (0, 0)),
            pl.BlockSpec((F, F), lambda i: (0, 0)),
            pl.BlockSpec((1, F), lambda i: (0, 0)),
        ],
        out_specs=pl.BlockSpec((EB, F), lambda i: (i, 0)),
        out_shape=jax.ShapeDtypeStruct((ecnt, F), jnp.float32),
    )(edge_attr, el2d, w1, b1.reshape(1, F), w2, b2.reshape(1, F))


_NC = 2
_NS = 16
_CH = 80


def _sc_segment(xh, wfilt, src, dst, eoff):
    N, F = xh.shape
    E = wfilt.shape[0]
    NW = _NC * _NS
    EW = E // NW
    NCH = EW // _CH
    NR = (N // _NS) // 8 * 8
    NREM = N - NR * _NS
    assert EW * NW == E and NCH * _CH == EW and N % 8 == 0 and NREM % 8 == 0

    mesh = plsc.VectorSubcoreMesh(core_axis_name="c", subcore_axis_name="s",
                                  num_cores=_NC, num_subcores=_NS)

    @functools.partial(
        pl.kernel,
        out_type=jax.ShapeDtypeStruct((_NC, N, F), jnp.float32),
        mesh=mesh,
        scratch_types=[
            pltpu.VMEM((_CH,), jnp.int32),
            pltpu.VMEM((_CH,), jnp.int32),
            pltpu.VMEM((_CH, F), jnp.float32),
            pltpu.VMEM((_CH, F), jnp.float32),
            pltpu.VMEM((_CH,), jnp.int32),
            pltpu.VMEM((_CH,), jnp.int32),
            pltpu.VMEM((_CH, F), jnp.float32),
            pltpu.VMEM((_CH, F), jnp.float32),
            pltpu.VMEM((16, F), jnp.float32),
            pltpu.VMEM_SHARED((N, F), jnp.float32),
            pltpu.SemaphoreType.DMA,
            pltpu.SemaphoreType.DMA,
            pltpu.SemaphoreType.DMA,
            pltpu.SemaphoreType.DMA,
        ],
    )
    def k(xh_hbm, w_hbm, src_hbm, dst_hbm, out_hbm,
          sidx, didx, rows, wrows, sidxb, didxb, rowsb, wrowsb,
          zbuf, agg_sh, sem, semb, sems, semsb):
        cid = lax.axis_index("c")
        sid = lax.axis_index("s")
        wid = cid * _NS + sid

        def zrow(r, carry):
            for j in range(F // 16):
                zbuf[r, pl.ds(j * 16, 16)] = jnp.zeros((16,), jnp.float32)
            return carry
        lax.fori_loop(0, 16, zrow, 0)

        def zcopy(c, carry):
            pltpu.sync_copy(zbuf, agg_sh.at[pl.ds(sid * NR + c * 16, 16)])
            return carry
        lax.fori_loop(0, NR // 16, zcopy, 0)
        if NREM:
            @pl.when(sid == _NS - 1)
            def _():
                pltpu.sync_copy(zbuf.at[pl.ds(0, NREM)],
                                agg_sh.at[pl.ds(_NS * NR, NREM)])
        plsc.subcore_barrier()

        base = wid * EW

        def mul_rows(r_ref, w_ref):
            @plsc.parallel_loop(0, _CH, unroll=2)
            def _(e):
                for j in range(F // 16):
                    s = pl.ds(j * 16, 16)
                    r_ref[e, s] = r_ref[e, s] * w_ref[e, s]

        def drain_scatters():
            pltpu.make_async_copy(rows, agg_sh.at[didx], sems).wait()
            pltpu.make_async_copy(rowsb, agg_sh.at[didxb], semsb).wait()

        def pair(k2, carry):
            offa = base + (2 * k2) * _CH
            offb = offa + _CH
            pltpu.sync_copy(src_hbm.at[pl.ds(eoff + offa, _CH)], sidx)
            pltpu.sync_copy(src_hbm.at[pl.ds(eoff + offb, _CH)], sidxb)

            @pl.when(k2 > 0)
            def _():
                drain_scatters()
            ga = pltpu.async_copy(xh_hbm.at[sidx], rows, sem)
            gb = pltpu.async_copy(xh_hbm.at[sidxb], rowsb, semb)
            pltpu.sync_copy(dst_hbm.at[pl.ds(eoff + offa, _CH)], didx)
            pltpu.sync_copy(dst_hbm.at[pl.ds(eoff + offb, _CH)], didxb)
            pltpu.sync_copy(w_hbm.at[pl.ds(offa, _CH)], wrows)
            pltpu.sync_copy(w_hbm.at[pl.ds(offb, _CH)], wrowsb)
            ga.wait()
            mul_rows(rows, wrows)
            pltpu.async_copy(rows, agg_sh.at[didx], sems, add=True)
            gb.wait()
            mul_rows(rowsb, wrowsb)
            pltpu.async_copy(rowsb, agg_sh.at[didxb], semsb, add=True)
            return carry
        lax.fori_loop(0, NCH // 2, pair, 0)
        if NCH >= 2:
            drain_scatters()

        if NCH % 2:
            off = base + (NCH - 1) * _CH
            pltpu.sync_copy(src_hbm.at[pl.ds(eoff + off, _CH)], sidx)
            pltpu.sync_copy(dst_hbm.at[pl.ds(eoff + off, _CH)], didx)
            pltpu.async_copy(xh_hbm.at[sidx], rows, sem).wait()
            pltpu.sync_copy(w_hbm.at[pl.ds(off, _CH)], wrows)
            mul_rows(rows, wrows)
            pltpu.sync_copy(rows, agg_sh.at[didx], add=True)

        plsc.subcore_barrier()
        pltpu.sync_copy(agg_sh.at[pl.ds(sid * NR, NR)],
                        out_hbm.at[cid, pl.ds(sid * NR, NR)])
        if NREM:
            @pl.when(sid == _NS - 1)
            def _():
                pltpu.sync_copy(agg_sh.at[pl.ds(_NS * NR, NREM)],
                                out_hbm.at[cid, pl.ds(_NS * NR, NREM)])

    return k(xh, wfilt, src, dst)


def _tail_body(agg_ref, aggb_ref, w2_ref, b2_ref, w_ref, b_ref, out_ref):
    a = (agg_ref[0] + agg_ref[1]) + (aggb_ref[0] + aggb_ref[1])
    t = _ssp(_dotT(a, w2_ref[...]) + b2_ref[...])
    out_ref[...] = _dotT(t, w_ref[...]) + b_ref[...]


def _compute_tail(agg2, agg2b, lin2_w, lin2_b, lin_w, lin_b):
    _, N, F = agg2.shape
    H = lin2_w.shape[0]
    NB = 2000 if N % 2000 == 0 else N
    grid = (N // NB,)
    return pl.pallas_call(
        _tail_body,
        grid=grid,
        in_specs=[
            pl.BlockSpec((2, NB, F), lambda i: (0, i, 0)),
            pl.BlockSpec((2, NB, F), lambda i: (0, i, 0)),
            pl.BlockSpec((H, F), lambda i: (0, 0)),
            pl.BlockSpec((1, H), lambda i: (0, 0)),
            pl.BlockSpec((H, H), lambda i: (0, 0)),
            pl.BlockSpec((1, H), lambda i: (0, 0)),
        ],
        out_specs=pl.BlockSpec((NB, H), lambda i: (i, 0)),
        out_shape=jax.ShapeDtypeStruct((N, H), jnp.float32),
    )(agg2, agg2b, lin2_w, lin2_b.reshape(1, H), lin_w, lin_b.reshape(1, H))


def kernel(x, edge_index, edge_length, edge_attr,
           mlp_w1, mlp_b1, mlp_w2, mlp_b2,
           lin1_w, lin2_w, lin2_b, lin_w, lin_b):
    src = edge_index[0].astype(jnp.int32)
    dst = edge_index[1].astype(jnp.int32)
    E = src.shape[0]
    E1 = 163840 if E == 320000 else E
    el2d = edge_length.reshape(E, 1)
    xh = _compute_xh(x, lin1_w)
    wf1 = _compute_filter(edge_attr, el2d, mlp_w1, mlp_b1,
                          mlp_w2, mlp_b2, 0, E1)
    p1 = _sc_segment(xh, wf1, src, dst, 0)
    if E1 < E:
        wf2 = _compute_filter(edge_attr, el2d, mlp_w1, mlp_b1,
                              mlp_w2, mlp_b2, E1, E - E1)
        p2 = _sc_segment(xh, wf2, src, dst, E1)
    else:
        p2 = jnp.zeros_like(p1)
    return _compute_tail(p1, p2, lin2_w, lin2_b, lin_w, lin_b)


# speedup vs baseline: 1.1781x; 1.1781x over previous
import functools
import math

import jax
import jax.numpy as jnp
from jax import lax
from jax.experimental import pallas as pl
from jax.experimental.pallas import tpu as pltpu
from jax.experimental.pallas import tpu_sc as plsc

CUTOFF = 10.0
LOG2 = math.log(2.0)


LOG2E = 1.4426950408889634

_EXP2C = (-0.0009475536455669522, 0.009210875560850372,
          -0.055299607117477535, 0.24017948948888648,
          -0.6931432016425585, 0.9999999452374342)
_LOG2C = (-0.00887469665141809, 0.05027750737105334,
          -0.13453425419918055, 0.2392316629725045,
          -0.34599601243315514, 0.47832354486763984,
          -0.7211146144033584, 1.4426867778259647,
          4.886358020325664e-08)


_COSC = (1.7295049974733663e-09, -2.7094490891269423e-07,
         2.4771702330757988e-05, -0.0013887904388839413,
         0.04166651546784112, -0.49999991826057477,
         0.9999999954856433)


def _poly(coeffs, x):
    acc = jnp.full_like(x, coeffs[0])
    for c in coeffs[1:]:
        acc = acc * x + c
    return acc


def _ssp(v):
    u = jnp.minimum(jnp.abs(v) * LOG2E, 30.0)
    n = u.astype(jnp.int32)
    f = u - n.astype(jnp.float32)
    scale = lax.bitcast_convert_type(
        lax.shift_left(127 - n, 23), jnp.float32)
    t = _poly(_EXP2C, f) * scale
    g = _poly(_LOG2C, t) * LOG2
    return jnp.maximum(v, 0.0) + g - LOG2


def _dotT(a, b):
    return lax.dot_general(a, b, (((1,), (1,)), ((), ())),
                           preferred_element_type=jnp.float32)


def _xh_body(x_ref, w_ref, out_ref):
    out_ref[...] = _dotT(x_ref[...], w_ref[...])


def _compute_xh(x, lin1_w):
    N, H = x.shape
    F = lin1_w.shape[0]
    return pl.pallas_call(
        _xh_body,
        out_shape=jax.ShapeDtypeStruct((N, F), jnp.float32),
    )(x, lin1_w)


def _filter_body(ea_ref, el_ref, w1_ref, b1_ref, w2_ref, b2_ref, out_ref):
    ea = ea_ref[...]
    el = el_ref[...]
    h = _ssp(_dotT(ea, w1_ref[...]) + b1_ref[...])
    w = _dotT(h, w2_ref[...]) + b2_ref[...]
    z = el * (math.pi / CUTOFF)
    c = 0.5 * (_poly(_COSC, z * z) + 1.0)
    c = c * (el <= CUTOFF).astype(jnp.float32) * (el >= 0.0).astype(jnp.float32)
    out_ref[...] = w * c


def _compute_filter(edge_attr, el2d, w1, b1, w2, b2):
    E, G = edge_attr.shape
    F = w2.shape[0]
    EB = 2560 if E % 2560 == 0 else E
    grid = (E // EB,)
    return pl.pallas_call(
        _filter_body,
        grid=grid,
        in_specs=[
            pl.BlockSpec((EB, G), lambda i: (i, 0)),
            pl.BlockSpec((EB, 1), lambda i: (i, 0)),
            pl.BlockSpec((F, G), lambda i: (0, 0)),
            pl.BlockSpec((1, F), lambda i: (0, 0)),
            pl.BlockSpec((F, F), lambda i: (0, 0)),
            pl.BlockSpec((1, F), lambda i: (0, 0)),
        ],
        out_specs=pl.BlockSpec((EB, F), lambda i: (i, 0)),
        out_shape=jax.ShapeDtypeStruct((E, F), jnp.float32),
    )(edge_attr, el2d, w1, b1.reshape(1, F), w2, b2.reshape(1, F))


_NC = 2
_NS = 16
_CH = 80


def _sc_segment(xh, wfilt, src, dst, eoff):
    N, F = xh.shape
    E = wfilt.shape[0]
    NW = _NC * _NS
    EW = E // NW
    NCH = EW // _CH
    NR = (N // _NS) // 8 * 8
    NREM = N - NR * _NS
    assert EW * NW == E and NCH * _CH == EW and N % 8 == 0 and NREM % 8 == 0

    mesh = plsc.VectorSubcoreMesh(core_axis_name="c", subcore_axis_name="s",
                                  num_cores=_NC, num_subcores=_NS)

    @functools.partial(
        pl.kernel,
        out_type=jax.ShapeDtypeStruct((_NC, N, F), jnp.float32),
        mesh=mesh,
        scratch_types=[
            pltpu.VMEM((_CH,), jnp.int32),
            pltpu.VMEM((_CH,), jnp.int32),
            pltpu.VMEM((_CH, F), jnp.float32),
            pltpu.VMEM((_CH, F), jnp.float32),
            pltpu.VMEM((_CH,), jnp.int32),
            pltpu.VMEM((_CH,), jnp.int32),
            pltpu.VMEM((_CH, F), jnp.float32),
            pltpu.VMEM((_CH, F), jnp.float32),
            pltpu.VMEM((16, F), jnp.float32),
            pltpu.VMEM_SHARED((N, F), jnp.float32),
            pltpu.SemaphoreType.DMA,
            pltpu.SemaphoreType.DMA,
            pltpu.SemaphoreType.DMA,
            pltpu.SemaphoreType.DMA,
        ],
    )
    def k(xh_hbm, w_hbm, src_hbm, dst_hbm, out_hbm,
          sidx, didx, rows, wrows, sidxb, didxb, rowsb, wrowsb,
          zbuf, agg_sh, sem, semb, sems, semsb):
        cid = lax.axis_index("c")
        sid = lax.axis_index("s")
        wid = cid * _NS + sid

        def zrow(r, carry):
            for j in range(F // 16):
                zbuf[r, pl.ds(j * 16, 16)] = jnp.zeros((16,), jnp.float32)
            return carry
        lax.fori_loop(0, 16, zrow, 0)

        def zcopy(c, carry):
            pltpu.sync_copy(zbuf, agg_sh.at[pl.ds(sid * NR + c * 16, 16)])
            return carry
        lax.fori_loop(0, NR // 16, zcopy, 0)
        if NREM:
            @pl.when(sid == _NS - 1)
            def _():
                pltpu.sync_copy(zbuf.at[pl.ds(0, NREM)],
                                agg_sh.at[pl.ds(_NS * NR, NREM)])
        plsc.subcore_barrier()

        base = wid * EW

        def mul_rows(r_ref, w_ref):
            @plsc.parallel_loop(0, _CH, unroll=2)
            def _(e):
                for j in range(F // 16):
                    s = pl.ds(j * 16, 16)
                    r_ref[e, s] = r_ref[e, s] * w_ref[e, s]

        def drain_scatters():
            pltpu.make_async_copy(rows, agg_sh.at[didx], sems).wait()
            pltpu.make_async_copy(rowsb, agg_sh.at[didxb], semsb).wait()

        def pair(k2, carry):
            offa = base + (2 * k2) * _CH
            offb = offa + _CH
            pltpu.sync_copy(src_hbm.at[pl.ds(eoff + offa, _CH)], sidx)
            pltpu.sync_copy(src_hbm.at[pl.ds(eoff + offb, _CH)], sidxb)

            @pl.when(k2 > 0)
            def _():
                drain_scatters()
            ga = pltpu.async_copy(xh_hbm.at[sidx], rows, sem)
            gb = pltpu.async_copy(xh_hbm.at[sidxb], rowsb, semb)
            pltpu.sync_copy(dst_hbm.at[pl.ds(eoff + offa, _CH)], didx)
            pltpu.sync_copy(dst_hbm.at[pl.ds(eoff + offb, _CH)], didxb)
            pltpu.sync_copy(w_hbm.at[pl.ds(offa, _CH)], wrows)
            pltpu.sync_copy(w_hbm.at[pl.ds(offb, _CH)], wrowsb)
            ga.wait()
            mul_rows(rows, wrows)
            pltpu.async_copy(rows, agg_sh.at[didx], sems, add=True)
            gb.wait()
            mul_rows(rowsb, wrowsb)
            pltpu.async_copy(rowsb, agg_sh.at[didxb], semsb, add=True)
            return carry
        lax.fori_loop(0, NCH // 2, pair, 0)
        if NCH >= 2:
            drain_scatters()

        if NCH % 2:
            off = base + (NCH - 1) * _CH
            pltpu.sync_copy(src_hbm.at[pl.ds(eoff + off, _CH)], sidx)
            pltpu.sync_copy(dst_hbm.at[pl.ds(eoff + off, _CH)], didx)
            pltpu.async_copy(xh_hbm.at[sidx], rows, sem).wait()
            pltpu.sync_copy(w_hbm.at[pl.ds(off, _CH)], wrows)
            mul_rows(rows, wrows)
            pltpu.sync_copy(rows, agg_sh.at[didx], add=True)

        plsc.subcore_barrier()
        pltpu.sync_copy(agg_sh.at[pl.ds(sid * NR, NR)],
                        out_hbm.at[cid, pl.ds(sid * NR, NR)])
        if NREM:
            @pl.when(sid == _NS - 1)
            def _():
                pltpu.sync_copy(agg_sh.at[pl.ds(_NS * NR, NREM)],
                                out_hbm.at[cid, pl.ds(_NS * NR, NREM)])

    return k(xh, wfilt, src, dst)


def _tail_body(a_ref, b_ref, c_ref, d_ref, w2_ref, b2_ref, w_ref, bo_ref,
               out_ref):
    a = ((a_ref[0] + a_ref[1]) + (b_ref[0] + b_ref[1]) +
         (c_ref[0] + c_ref[1]) + (d_ref[0] + d_ref[1]))
    t = _ssp(_dotT(a, w2_ref[...]) + b2_ref[...])
    out_ref[...] = _dotT(t, w_ref[...]) + bo_ref[...]


def _compute_tail(parts, lin2_w, lin2_b, lin_w, lin_b):
    _, N, F = parts[0].shape
    H = lin2_w.shape[0]
    NB = 2000 if N % 2000 == 0 else N
    grid = (N // NB,)
    pspec = pl.BlockSpec((2, NB, F), lambda i: (0, i, 0))
    return pl.pallas_call(
        _tail_body,
        grid=grid,
        in_specs=[
            pspec, pspec, pspec, pspec,
            pl.BlockSpec((H, F), lambda i: (0, 0)),
            pl.BlockSpec((1, H), lambda i: (0, 0)),
            pl.BlockSpec((H, H), lambda i: (0, 0)),
            pl.BlockSpec((1, H), lambda i: (0, 0)),
        ],
        out_specs=pl.BlockSpec((NB, H), lambda i: (i, 0)),
        out_shape=jax.ShapeDtypeStruct((N, H), jnp.float32),
    )(*parts, lin2_w, lin2_b.reshape(1, H), lin_w, lin_b.reshape(1, H))


def kernel(x, edge_index, edge_length, edge_attr,
           mlp_w1, mlp_b1, mlp_w2, mlp_b2,
           lin1_w, lin2_w, lin2_b, lin_w, lin_b):
    src = edge_index[0].astype(jnp.int32)
    dst = edge_index[1].astype(jnp.int32)
    E = src.shape[0]
    if E == 320000:
        bounds = [0, 81920, 163840, 245760, 320000]
    else:
        bounds = [0, E]
    xh = _compute_xh(x, lin1_w)
    partials = []
    for lo, hi in zip(bounds[:-1], bounds[1:]):
        wf = _compute_filter(edge_attr[lo:hi],
                             edge_length[lo:hi].reshape(hi - lo, 1),
                             mlp_w1, mlp_b1, mlp_w2, mlp_b2)
        partials.append(_sc_segment(xh, wf, src[lo:hi], dst[lo:hi], 0))
    while len(partials) < 4:
        partials.append(jnp.zeros_like(partials[0]))
    return _compute_tail(partials, lin2_w, lin2_b, lin_w, lin_b)
